# baseline (speedup 1.0000x reference)
import jax
import jax.numpy as jnp
from jax.experimental import pallas as pl
from jax.experimental.pallas import tpu as pltpu

B, N_PER, E_PER, D, R_PER, C_DIM = 8, 1250, 40000, 128, 625, 4
E_SUB, E_LANE = 40, 1000


def _merge_body(x_ref, e_ref, ox_ref, oe_ref):
    b = pl.program_id(0)
    ox_ref[...] = x_ref[...]
    oe_ref[:, 0] = e_ref[0] + b * N_PER


def kernel(x, shift, shape, coupling, edge_index):
    edges4 = edge_index.reshape(B, 2, E_SUB, E_LANE)

    out_x, out_e = pl.pallas_call(
        _merge_body,
        grid=(B,),
        in_specs=[
            pl.BlockSpec((1, N_PER, D), lambda b: (b, 0, 0)),
            pl.BlockSpec((1, 2, E_SUB, E_LANE), lambda b: (b, 0, 0, 0)),
        ],
        out_specs=[
            pl.BlockSpec((1, N_PER, D), lambda b: (b, 0, 0)),
            pl.BlockSpec((2, 1, E_SUB, E_LANE), lambda b: (0, b, 0, 0)),
        ],
        out_shape=[
            jax.ShapeDtypeStruct((B, N_PER, D), jnp.float32),
            jax.ShapeDtypeStruct((2, B, E_SUB, E_LANE), jnp.int32),
        ],
        compiler_params=pltpu.CompilerParams(
            dimension_semantics=("parallel",),
        ),
    )(x, edges4)

    return (
        out_x.reshape(B * N_PER, D),
        out_e.reshape(2, B * E_PER),
        shift.reshape(B * R_PER),
        shape.reshape(B * R_PER),
        coupling.reshape(B * R_PER, C_DIM),
    )


# speedup vs baseline: 1.2144x; 1.2144x over previous
import jax
import jax.numpy as jnp
from jax.experimental import pallas as pl
from jax.experimental.pallas import tpu as pltpu

B, N_PER, E_PER, D, R_PER, C_DIM = 8, 1250, 40000, 128, 625, 4
E_SUB, E_LANE = 40, 1000


def _edge_body(e_ref, oe_ref):
    b = pl.program_id(0)
    oe_ref[:, 0] = e_ref[0] + b * N_PER


def kernel(x, shift, shape, coupling, edge_index):
    edges4 = edge_index.reshape(B, 2, E_SUB, E_LANE)

    out_e = pl.pallas_call(
        _edge_body,
        grid=(B,),
        in_specs=[pl.BlockSpec((1, 2, E_SUB, E_LANE), lambda b: (b, 0, 0, 0))],
        out_specs=[pl.BlockSpec((2, 1, E_SUB, E_LANE), lambda b: (0, b, 0, 0))],
        out_shape=[jax.ShapeDtypeStruct((2, B, E_SUB, E_LANE), jnp.int32)],
        compiler_params=pltpu.CompilerParams(
            dimension_semantics=("arbitrary",),
        ),
    )(edges4)[0]

    return (
        x.reshape(B * N_PER, D),
        out_e.reshape(2, B * E_PER),
        shift.reshape(B * R_PER),
        shape.reshape(B * R_PER),
        coupling.reshape(B * R_PER, C_DIM),
    )
